# Initial kernel scaffold; baseline (speedup 1.0000x reference)
import jax
import jax.numpy as jnp
from jax.experimental import pallas as pl


def kernel(x, edge_index, batch, params):
    raise NotImplementedError("write your pallas kernel here")


# speedup vs baseline: 3.2334x; 3.2334x over previous
import functools

import jax
import jax.numpy as jnp
import numpy as np
from jax import lax
from jax.experimental import pallas as pl
from jax.experimental.pallas import tpu as pltpu
from jax.experimental.pallas import tpu_sc as plsc

N = 10000
E = 320000
D = 128
NUM_GRAPHS = 128
GAMMA = 0.4

_NC = 2
_NS = 16
_NW = _NC * _NS
_C = 80
_NPAD = 10240
_RPT = _NPAD // _NS
_ZR = 128

_CNTS = [126] * 11 + [123] * 4 + [122]
_BASES = [sum(_CNTS[:i]) for i in range(_NS)]


def _edge_agg_body(h_hbm, src_hbm, dst_hbm, fn_hbm, out_hbm,
                   src_v, dst_v, rows_v, zbuf_v, fn_v,
                   pb_v, idx1_v, acc_sh, sem):
    cid = lax.axis_index("c")
    sid = lax.axis_index("s")

    def zrow(r, carry):
        for q in range(D // 16):
            zbuf_v[r, pl.ds(q * 16, 16)] = jnp.zeros((16,), jnp.float32)
        return carry
    lax.fori_loop(0, _ZR, zrow, 0)
    for k in range(_RPT // _ZR):
        pltpu.sync_copy(zbuf_v, acc_sh.at[pl.ds(sid * _RPT + k * _ZR, _ZR)])

    base_chunk = jnp.where(sid < 11, sid * 126, 1386 + (sid - 11) * 123)
    nch = jnp.where(sid < 11, 126, jnp.where(sid < 15, 123, 122))
    ebase = cid * (E // _NC) + base_chunk * _C

    wid = cid * _NS + sid
    pltpu.sync_copy(fn_hbm.at[pl.ds(pl.multiple_of(wid * 16, 16), 16)], fn_v)
    fnvec = fn_v[...]
    dummy_row = N + 8 * sid
    plsc.subcore_barrier()

    def chunk(i, carry):
        off = pl.multiple_of(ebase + i * _C, _C)
        pltpu.sync_copy(src_hbm.at[pl.ds(off, _C)], src_v)
        pltpu.sync_copy(dst_hbm.at[pl.ds(off, _C)], dst_v)
        for q in range(_C // 16):
            v = dst_v[pl.ds(q * 16, 16)]
            dst_v[pl.ds(q * 16, 16)] = jnp.where(v == fnvec,
                                                 jnp.int32(dummy_row), v)
        pltpu.async_copy(h_hbm.at[src_v], rows_v, sem).wait()
        pltpu.sync_copy(rows_v, acc_sh.at[dst_v], add=True)
        return carry
    lax.fori_loop(0, nch, chunk, 0)
    plsc.subcore_barrier()

    pltpu.sync_copy(acc_sh.at[pl.ds(pl.multiple_of(dummy_row, 8), 16)], pb_v)
    trash = jnp.full((16,), N + 232, jnp.int32)
    tgt = jnp.where(fnvec >= 0, fnvec, trash)
    lane0 = lax.broadcasted_iota(jnp.int32, (16,), 0) == 0
    idx1_v[...] = jnp.where(lane0, tgt, trash)
    pltpu.sync_copy(pb_v, acc_sh.at[idx1_v], add=True)
    plsc.subcore_barrier()

    pltpu.sync_copy(acc_sh.at[pl.ds(sid * _RPT, _RPT)],
                    out_hbm.at[cid, pl.ds(sid * _RPT, _RPT)])


@jax.jit
def _edge_agg(h, src_s, dst_s, fn_bcast):
    mesh = plsc.VectorSubcoreMesh(core_axis_name="c", subcore_axis_name="s")
    k = pl.kernel(
        _edge_agg_body,
        out_type=jax.ShapeDtypeStruct((_NC, _NPAD, D), jnp.float32),
        mesh=mesh,
        scratch_types=[
            pltpu.VMEM((_C,), jnp.int32),
            pltpu.VMEM((_C,), jnp.int32),
            pltpu.VMEM((_C, D), jnp.float32),
            pltpu.VMEM((_ZR, D), jnp.float32),
            pltpu.VMEM((16,), jnp.int32),
            pltpu.VMEM((16, D), jnp.float32),
            pltpu.VMEM((16,), jnp.int32),
            pltpu.VMEM_SHARED((_NPAD, D), jnp.float32),
            pltpu.SemaphoreType.DMA,
        ],
    )
    return k(h, src_s, dst_s, fn_bcast)


_INV_N = float(np.float32(1.0) / np.float32(N))


def _fold8(acc):
    a = acc[0:4] + acc[4:8]
    a = a[0:2] + a[2:4]
    return a[0:1] + a[1:2]


def _bn_stats(ref):
    nch = N // 8
    c = ref.shape[1]

    def mstep(i, acc):
        return acc + ref[pl.ds(i * 8, 8), :]
    macc = lax.fori_loop(0, nch, mstep, jnp.zeros((8, c), jnp.float32))
    m = _fold8(macc) * _INV_N

    def vstep_lo(i, acc):
        d = ref[pl.ds(i * 8, 8), :] - m
        return acc + d * d

    def vstep_hi(i, acc):
        d = ref[pl.ds((nch // 2 + i) * 8, 8), :] - m
        return acc + d * d
    va = lax.fori_loop(0, nch // 2, vstep_lo, jnp.zeros((8, c), jnp.float32))
    vb = lax.fori_loop(0, nch // 2, vstep_hi, jnp.zeros((8, c), jnp.float32))
    v = (_fold8(va) + _fold8(vb)) * _INV_N
    return m, v


def _bn_small(x):
    m = jnp.mean(x, axis=0, keepdims=True)
    v = jnp.mean((x - m) * (x - m), axis=0, keepdims=True)
    return (x - m) / jnp.sqrt(v + 1e-5)


def _enc_body(x_ref, w1_ref, b1_ref, w2_ref, b2_ref, o1_ref, o2_ref):
    x = x_ref[...]
    o1_ref[...] = x @ w1_ref[...] + b1_ref[...]
    o2_ref[...] = x @ w2_ref[...] + b2_ref[...]


@jax.jit
def _encode(x, w1, b1, w2, b2):
    return pl.pallas_call(
        _enc_body,
        out_shape=(jax.ShapeDtypeStruct((N, D), jnp.float32),
                   jax.ShapeDtypeStruct((N, D), jnp.float32)),
    )(x, w1, b1.reshape(1, D), w2, b2.reshape(1, D))


def _gin_body(h_ref, agg_ref, w1_ref, b1_ref, w2_ref, b2_ref, eps_ref,
              out_ref, u_scr, v_scr, *, last):
    h = h_ref[...]
    z = (1.0 + eps_ref[0, 0]) * h + (agg_ref[0, :N] + agg_ref[1, :N])
    u_scr[...] = z @ w1_ref[...] + b1_ref[...]
    m, v = _bn_stats(u_scr)
    r = jnp.maximum((u_scr[...] - m) / jnp.sqrt(v + 1e-5), 0.0)
    v_scr[...] = r @ w2_ref[...] + b2_ref[...]
    m2, v2 = _bn_stats(v_scr)
    w = (v_scr[...] - m2) / jnp.sqrt(v2 + 1e-5)
    if not last:
        w = jnp.maximum(w, 0.0)
    out_ref[...] = w + h


@functools.partial(jax.jit, static_argnames=("last",))
def _gin_dense(h, agg2, p, last):
    body = functools.partial(_gin_body, last=last)
    return pl.pallas_call(
        body,
        out_shape=jax.ShapeDtypeStruct((N, D), jnp.float32),
        scratch_shapes=[pltpu.VMEM((N, 2 * D), jnp.float32),
                        pltpu.VMEM((N, D), jnp.float32)],
    )(h, agg2, p["W1"], p["b1"].reshape(1, -1), p["W2"],
      p["b2"].reshape(1, -1), p["eps"].reshape(1, 1))


def _head_body(h5_ref, h1_ref, h2_ref, h3_ref, h4_ref, xr_ref,
               batch_ref, u_ref, perm_ref,
               gw1_ref, gb1_ref, gw2_ref, gb2_ref,
               pw1_ref, pb1_ref, pw2_ref, pb2_ref,
               pred_rem_ref, loss_reg_ref, preds_ref, loss_con_ref, g_scr):
    xr = xr_ref[...]
    g_scr[...] = xr @ gw1_ref[...] + gb1_ref[...]
    gm, gv = _bn_stats(g_scr)
    gu = jnp.maximum((g_scr[...] - gm) / jnp.sqrt(gv + 1e-5), 0.0)
    glog = gu @ gw2_ref[...] + gb2_ref[...]
    g = -jnp.log(-jnp.log(u_ref[...]))
    a = glog + g
    gate = 1.0 / (1.0 + jnp.exp(a[:, 0:1] - a[:, 1:2]))
    gate_row = gate.reshape(1, N)

    gids = lax.broadcasted_iota(jnp.int32, (NUM_GRAPHS, N), 0)
    onehot = (gids == batch_ref[...]).astype(jnp.float32)
    og = onehot * gate_row
    oe = onehot - og
    cnt = jnp.maximum(jnp.sum(onehot, axis=1, keepdims=True), 1.0)

    h5 = h5_ref[...]
    h_r = (og @ h5) / cnt
    h_env = (oe @ h5) / cnt
    r_num = jnp.sum(og, axis=1, keepdims=True) + 1e-8
    e_num = jnp.sum(oe, axis=1, keepdims=True) + 1e-8
    loss_reg_ref[...] = jnp.mean(
        jnp.abs(r_num / (r_num + e_num) - GAMMA)).reshape(1, 1)

    def mlp_pred(xin):
        hh = jnp.maximum(_bn_small(xin @ pw1_ref[...] + pb1_ref[...]), 0.0)
        return hh @ pw2_ref[...] + pb2_ref[...]

    pids = lax.broadcasted_iota(jnp.int32, (5, NUM_GRAPHS, NUM_GRAPHS), 2)
    env_in = (h1_ref[...], h2_ref[...], h3_ref[...], h4_ref[...])
    for i in range(4):
        envp = (oe @ env_in[i]) / cnt
        poh = (perm_ref[i].reshape(NUM_GRAPHS, 1) == pids[i]).astype(jnp.float32)
        preds_ref[i] = mlp_pred(h_r + poh @ envp)
    poh = (perm_ref[4].reshape(NUM_GRAPHS, 1) == pids[4]).astype(jnp.float32)
    preds_ref[4] = mlp_pred(h_r + poh @ h_env)
    pred_rem_ref[...] = mlp_pred(h_r)

    x = h_r
    x_aug = h_r + h_env
    x_cp = h_env
    xa = jnp.sqrt(jnp.sum(x * x, axis=1, keepdims=True))
    xaa = jnp.sqrt(jnp.sum(x_aug * x_aug, axis=1, keepdims=True))
    xca = jnp.sqrt(jnp.sum(x_cp * x_cp, axis=1, keepdims=True))
    T = 0.2
    pos = jnp.exp(jnp.sum(x * x_aug, axis=1, keepdims=True)
                  / (xa * xaa + 1e-8) / T)
    simcp = jnp.exp((x @ x_cp.T) / (xa * xca.reshape(1, NUM_GRAPHS) + 1e-8) / T)
    loss2 = pos / (jnp.sum(simcp, axis=1, keepdims=True) + pos)
    loss_con_ref[...] = -jnp.mean(jnp.log(loss2)).reshape(1, 1)


@jax.jit
def _head(h5, h1, h2, h3, h4, xr, batch_row, u, perms, gate_p, pred_p):
    return pl.pallas_call(
        _head_body,
        out_shape=(jax.ShapeDtypeStruct((NUM_GRAPHS, 1), jnp.float32),
                   jax.ShapeDtypeStruct((1, 1), jnp.float32),
                   jax.ShapeDtypeStruct((5, NUM_GRAPHS, 1), jnp.float32),
                   jax.ShapeDtypeStruct((1, 1), jnp.float32)),
        scratch_shapes=[pltpu.VMEM((N, 2 * D), jnp.float32)],
    )(h5, h1, h2, h3, h4, xr, batch_row, u, perms,
      gate_p["W1"], gate_p["b1"].reshape(1, -1),
      gate_p["W2"], gate_p["b2"].reshape(1, -1),
      pred_p["W1"], pred_p["b1"].reshape(1, -1),
      pred_p["W2"], pred_p["b2"].reshape(1, -1))


def kernel(x, edge_index, batch, params):
    order = jnp.argsort(edge_index[1], stable=True)
    src = edge_index[0][order]
    dst = edge_index[1][order]

    bpos = jnp.asarray([c * (E // _NC) + b * _C
                        for c in range(_NC) for b in _BASES], jnp.int32)
    d0 = dst[bpos]
    dm1 = dst[jnp.maximum(bpos - 1, 0)]
    is_shared = jnp.logical_and(bpos % (E // _NC) != 0, dm1 == d0)
    fn_bcast = jnp.repeat(jnp.where(is_shared, d0, -1).astype(jnp.int32), 16)

    h, xr = _encode(x, params["enc"]["W"], params["enc"]["b"],
                    params["renc"]["W"], params["renc"]["b"])

    glayers = params["glayers"]
    h_list = [h]
    for l in range(len(glayers)):
        agg2 = _edge_agg(h_list[l], src, dst, fn_bcast)
        h_new = _gin_dense(h_list[l], agg2, glayers[l],
                           last=(l == len(glayers) - 1))
        h_list.append(h_new)

    rlayers = params["rlayers"]
    for l in range(len(rlayers)):
        agg2 = _edge_agg(xr, src, dst, fn_bcast)
        xr = _gin_dense(xr, agg2, rlayers[l], last=(l == len(rlayers) - 1))

    gkey = jax.random.fold_in(jax.random.key(7), 0)
    u = jax.random.uniform(gkey, (N, 2), jnp.float32, 1e-10, 1.0)
    perms = jnp.stack(
        [jax.random.permutation(jax.random.fold_in(jax.random.key(11), i),
                                NUM_GRAPHS)
         for i in (0, 1, 2, 3, 99)]).astype(jnp.int32)

    pred_rem, loss_reg, preds, loss_con = _head(
        h_list[5], h_list[1], h_list[2], h_list[3], h_list[4], xr,
        batch.reshape(1, N).astype(jnp.int32), u, perms,
        params["gate"], params["pred"])

    return (pred_rem, loss_reg.reshape(()), preds, loss_con.reshape(()))
